# Initial kernel scaffold; baseline (speedup 1.0000x reference)
import jax
import jax.numpy as jnp
from jax.experimental import pallas as pl


def kernel(x, edge_index, batch, params):
    raise NotImplementedError("write your pallas kernel here")


# speedup vs baseline: 5.7421x; 5.7421x over previous
import functools

import jax
import jax.numpy as jnp
from jax import lax
from jax.experimental import pallas as pl
from jax.experimental.pallas import tpu as pltpu
from jax.experimental.pallas import tpu_sc as plsc

N_NODES = 10000
N_EDGES = 320000
D = 128
N_GRAPHS = 64

NC = 2
NS = 16
NW = NC * NS
CHUNK = 128
N_CHUNKS = N_EDGES // CHUNK
CH_PER_W = -(-N_CHUNKS // NW)
ACC_ROWS = 10112
ROWS_PER_TILE = ACC_ROWS // NS


def _seg_sum_sc(h, src2, dst2):
    mesh = plsc.VectorSubcoreMesh(
        core_axis_name="c", subcore_axis_name="s", num_cores=NC, num_subcores=NS
    )

    @functools.partial(
        pl.kernel,
        out_type=jax.ShapeDtypeStruct((NC, ACC_ROWS, D), jnp.float32),
        mesh=mesh,
        scratch_types=[
            pltpu.VMEM((CHUNK,), jnp.int32),
            pltpu.VMEM((CHUNK,), jnp.int32),
            pltpu.VMEM((CHUNK, D), jnp.float32),
            pltpu.VMEM_SHARED((ACC_ROWS, D), jnp.float32),
            pltpu.SemaphoreType.DMA,
        ],
    )
    def body(h_hbm, src_hbm, dst_hbm, out_hbm, src_v, dst_v, rows_v, acc, sem):
        cid = lax.axis_index("c")
        sid = lax.axis_index("s")
        wid = sid * NC + cid

        def zrow(i, _):
            def zcol(k, _):
                rows_v[i, pl.ds(k * 16, 16)] = jnp.zeros((16,), jnp.float32)
                return 0

            lax.fori_loop(0, D // 16, zcol, 0)
            return 0

        lax.fori_loop(0, CHUNK, zrow, 0)

        zbase = sid * ROWS_PER_TILE
        n_full = ROWS_PER_TILE // CHUNK
        for t in range(n_full):
            pltpu.sync_copy(rows_v, acc.at[pl.ds(zbase + t * CHUNK, CHUNK)])
        rem = ROWS_PER_TILE - n_full * CHUNK
        if rem:
            pltpu.sync_copy(
                rows_v.at[pl.ds(0, rem)],
                acc.at[pl.ds(zbase + n_full * CHUNK, rem)],
            )

        plsc.subcore_barrier()

        def chunk_body(j, _):
            r = wid + j * NW

            @pl.when(r < N_CHUNKS)
            def _():
                pltpu.sync_copy(src_hbm.at[r], src_v)
                pltpu.sync_copy(dst_hbm.at[r], dst_v)
                pltpu.async_copy(h_hbm.at[src_v], rows_v, sem).wait()
                pltpu.sync_copy(rows_v, acc.at[dst_v], add=True)

            return 0

        lax.fori_loop(0, CH_PER_W, chunk_body, 0)

        plsc.subcore_barrier()

        pltpu.sync_copy(
            acc.at[pl.ds(zbase, ROWS_PER_TILE)],
            out_hbm.at[cid, pl.ds(zbase, ROWS_PER_TILE)],
        )

    return body(h, src2, dst2)


_BLK = 1000
_GRID = N_NODES // _BLK


def _mlp_body(eps_s, h_ref, parts_ref, w1t, b1, w2t, b2, g, b, rm, rv, out_ref):
    t = (1.0 + eps_s[0, 0]) * h_ref[...] + parts_ref[0] + parts_ref[1]
    a = jnp.dot(t, w1t[...], preferred_element_type=jnp.float32) + b1[...]
    a = jnp.maximum(a, 0.0)
    a = jnp.dot(a, w2t[...], preferred_element_type=jnp.float32) + b2[...]
    a = jnp.maximum(a, 0.0)
    scale = g[...] * lax.rsqrt(rv[...] + 1e-5)
    out_ref[...] = a * scale + (b[...] - rm[...] * scale)


def _mlp_tc(h, parts, p):
    vec = lambda v: v.reshape(1, D)
    full = pl.BlockSpec((D, D), lambda i: (0, 0))
    vspec = pl.BlockSpec((1, D), lambda i: (0, 0))
    return pl.pallas_call(
        _mlp_body,
        grid=(_GRID,),
        in_specs=[
            pl.BlockSpec(memory_space=pltpu.SMEM),
            pl.BlockSpec((_BLK, D), lambda i: (i, 0)),
            pl.BlockSpec((NC, _BLK, D), lambda i: (0, i, 0)),
            full, vspec, full, vspec, vspec, vspec, vspec, vspec,
        ],
        out_specs=pl.BlockSpec((_BLK, D), lambda i: (i, 0)),
        out_shape=jax.ShapeDtypeStruct((N_NODES, D), jnp.float32),
    )(
        p["eps"].reshape(1, 1),
        h,
        parts,
        p["W1"].T,
        vec(p["b1"]),
        p["W2"].T,
        vec(p["b2"]),
        vec(p["bn_g"]),
        vec(p["bn_b"]),
        vec(p["bn_rm"]),
        vec(p["bn_rv"]),
    )


def _head_body(h_ref, batch_ref, w1t, b1, w2t, b2, out_ref):
    ids = lax.broadcasted_iota(jnp.int32, (N_NODES, N_GRAPHS), 1)
    m = (batch_ref[...] == ids).astype(jnp.float32)
    dn = (((0,), (0,)), ((), ()))
    sums = lax.dot_general(m, h_ref[...], dn, preferred_element_type=jnp.float32)
    counts = lax.dot_general(
        m, jnp.ones((N_NODES, 1), jnp.float32), dn,
        preferred_element_type=jnp.float32,
    )
    pooled = sums / jnp.maximum(counts, 1.0)
    a = jnp.dot(pooled, w1t[...], preferred_element_type=jnp.float32) + b1[...]
    a = jnp.maximum(a, 0.0)
    z = jnp.dot(a, w2t[...], preferred_element_type=jnp.float32) + b2[...]
    zmax = jnp.max(z, axis=1, keepdims=True)
    e = jnp.exp(z - zmax)
    lse = jnp.log(jnp.sum(e, axis=1, keepdims=True))
    out_ref[...] = z - zmax - lse


def _head_tc(h, batch, params):
    return pl.pallas_call(
        _head_body,
        out_shape=jax.ShapeDtypeStruct((N_GRAPHS, 10), jnp.float32),
    )(
        h,
        batch.reshape(N_NODES, 1),
        params["lin1_W"].T,
        params["lin1_b"].reshape(1, D),
        params["lin2_W"].T,
        params["lin2_b"].reshape(1, 10),
    )


def kernel(x, edge_index, batch, params):
    src2 = edge_index[0].astype(jnp.int32).reshape(N_CHUNKS, CHUNK)
    dst2 = edge_index[1].astype(jnp.int32).reshape(N_CHUNKS, CHUNK)
    h = x
    layer_params = [params["conv1"]] + list(params["convs"])
    for p in layer_params:
        parts = _seg_sum_sc(h, src2, dst2)
        h = _mlp_tc(h, parts, p)
    return _head_tc(h, batch.astype(jnp.int32), params)
